# baseline (speedup 1.0000x reference)
import functools

import jax
import jax.numpy as jnp
from jax import lax
from jax.experimental import pallas as pl
from jax.experimental.pallas import tpu as pltpu
from jax.experimental.pallas import tpu_sc as plsc

NUM_CORES = 2
NUM_SUBCORES = 16
NUM_WORKERS = NUM_CORES * NUM_SUBCORES

S = 4096
T = 200
B = S * T
D = 64
HALF = 32
E = 2 * B

STREAM = 100
G = 8
GROUP = G * STREAM

E_PER_W = E // NUM_WORKERS
N_GROUPS = E_PER_W // GROUP
ROWS_PER_W = E_PER_W // STREAM

NBUF = 2

_mesh = plsc.VectorSubcoreMesh(core_axis_name="c", subcore_axis_name="s")


@functools.partial(
    pl.kernel,
    mesh=_mesh,
    out_type=jax.ShapeDtypeStruct((E // STREAM, STREAM, HALF), jnp.float32),
    scratch_types=[
        pltpu.VMEM((ROWS_PER_W, STREAM), jnp.int32),
        pltpu.VMEM((G, STREAM, HALF), jnp.float32),
        pltpu.VMEM((G, STREAM, HALF), jnp.float32),
        pltpu.SemaphoreType.DMA,
        pltpu.SemaphoreType.DMA,
        pltpu.SemaphoreType.DMA,
        pltpu.SemaphoreType.DMA,
    ],
    compiler_params=pltpu.CompilerParams(use_tc_tiling_on_sc=False),
)
def _emb_lookup(idx_hbm, table_hbm, out_hbm, idx_v, rows0, rows1,
                gat_sem0, gat_sem1, wb_sem0, wb_sem1):
    rows = (rows0, rows1)
    gat_sem = (gat_sem0, gat_sem1)
    wb_sem = (wb_sem0, wb_sem1)

    wid = lax.axis_index("s") * NUM_CORES + lax.axis_index("c")
    row_base = wid * ROWS_PER_W

    pltpu.async_copy(
        idx_hbm.at[pl.ds(row_base, ROWS_PER_W)], idx_v, gat_sem0
    ).wait()

    def streams(g, b):
        out = []
        for j in range(G):
            out.append((
                table_hbm.at[idx_v.at[g * G + j]],
                rows[b].at[j],
                gat_sem[b],
            ))
        return out

    def fire(g, b):
        for src, dst, sem in streams(g, b):
            pltpu.async_copy(src, dst, sem)

    def drain_gathers(g, b):
        for src, dst, sem in streams(g, b):
            pltpu.make_async_copy(src, dst, sem).wait()

    for b in range(NBUF):
        fire(b, b)

    def body(k, _):
        for b in range(NBUF):
            g_done = k * NBUF + b
            drain_gathers(g_done, b)
            pltpu.async_copy(
                rows[b], out_hbm.at[pl.ds(row_base + g_done * G, G)],
                wb_sem[b],
            ).wait()
            fire(g_done + NBUF, b)
        return ()

    lax.fori_loop(0, N_GROUPS // NBUF - 1, body, (), unroll=False)

    for b in range(NBUF):
        g_done = N_GROUPS - NBUF + b
        drain_gathers(g_done, b)
        pltpu.async_copy(
            rows[b], out_hbm.at[pl.ds(row_base + g_done * G, G)],
            wb_sem[b],
        ).wait()


def kernel(sentences_indices, table):
    table_half = table.reshape(2 * 1000000, HALF)
    idx = sentences_indices.reshape(-1).astype(jnp.int32)
    ent = jnp.stack([2 * idx, 2 * idx + 1], axis=-1).reshape(
        E // STREAM, STREAM)
    out3 = _emb_lookup(ent, table_half)
    return out3.reshape(S, T, D)


# speedup vs baseline: 1.1701x; 1.1701x over previous
import functools

import jax
import jax.numpy as jnp
from jax import lax
from jax.experimental import pallas as pl
from jax.experimental.pallas import tpu as pltpu
from jax.experimental.pallas import tpu_sc as plsc

NUM_CORES = 2
NUM_SUBCORES = 16
NUM_WORKERS = NUM_CORES * NUM_SUBCORES

S = 4096
T = 200
D = 64
SB = S // 128
L = 16

GT = 2
NBUF = 2
N_SLOTS = T // GT

_mesh = plsc.VectorSubcoreMesh(core_axis_name="c", subcore_axis_name="s")


@functools.partial(
    pl.kernel,
    mesh=_mesh,
    out_type=jax.ShapeDtypeStruct((T, D // 8, SB, 1024), jnp.float32),
    scratch_types=[
        pltpu.VMEM((T, 128), jnp.int32),
        pltpu.VMEM((GT, 128, D), jnp.float32),
        pltpu.VMEM((GT, 128, D), jnp.float32),
        pltpu.VMEM((GT, D // 8, 1024), jnp.float32),
        pltpu.VMEM((GT, D // 8, 1024), jnp.float32),
        pltpu.SemaphoreType.DMA,
        pltpu.SemaphoreType.DMA,
        pltpu.SemaphoreType.DMA,
        pltpu.SemaphoreType.DMA,
    ],
    compiler_params=pltpu.CompilerParams(
        use_tc_tiling_on_sc=False,
        needs_layout_passes=False,
    ),
)
def _emb_lookup(idx_hbm, table_hbm, out_hbm, idx_v, rows0, rows1,
                tp0, tp1, gat_sem0, gat_sem1, wb_sem0, wb_sem1):
    rows = (rows0, rows1)
    tp = (tp0, tp1)
    gat_sem = (gat_sem0, gat_sem1)
    wb_sem = (wb_sem0, wb_sem1)

    wid = lax.axis_index("s") * NUM_CORES + lax.axis_index("c")

    pltpu.async_copy(idx_hbm.at[:, wid, :], idx_v, gat_sem0).wait()

    def gathers(g, b):
        out = []
        for tt in range(GT):
            out.append((
                table_hbm.at[idx_v.at[g * GT + tt]],
                rows[b].at[tt],
                gat_sem[b],
            ))
        return out

    def fire(g, b):
        for src, dst, sem in gathers(g, b):
            pltpu.async_copy(src, dst, sem)

    def drain(g, b):
        for src, dst, sem in gathers(g, b):
            pltpu.make_async_copy(src, dst, sem).wait()

    iota = lax.iota(jnp.int32, L)

    def transpose(b):
        def dr_body(dr, _):
            base = dr * 128
            for tt in range(GT):
                for db in range(D // 8):
                    dvec = iota * 0 + (db * 8) + dr
                    for c in range(8):
                        svec = iota + c * L
                        v = plsc.load_gather(rows[b].at[tt], [svec, dvec])
                        tp[b][tt, db, pl.ds(base + c * L, L)] = v
            return ()

        lax.fori_loop(0, 8, dr_body, (), unroll=False)

    def wb_copies(g, b):
        out = []
        for tt in range(GT):
            for db in range(D // 8):
                out.append((
                    tp[b].at[tt, db],
                    out_hbm.at[g * GT + tt, db, wid],
                    wb_sem[b],
                ))
        return out

    def fire_wb(g, b):
        for src, dst, sem in wb_copies(g, b):
            pltpu.async_copy(src, dst, sem)

    def drain_wb(g, b):
        for src, dst, sem in wb_copies(g, b):
            pltpu.make_async_copy(src, dst, sem).wait()

    for b in range(NBUF):
        fire(b, b)

    def body(k, _):
        for b in range(NBUF):
            g = k * NBUF + b
            drain(g, b)
            @pl.when(k > 0)
            def _():
                drain_wb(g - NBUF, b)
            transpose(b)
            fire_wb(g, b)
            fire(g + NBUF, b)
        return ()

    lax.fori_loop(0, N_SLOTS // NBUF - 1, body, (), unroll=False)

    for b in range(NBUF):
        g = N_SLOTS - NBUF + b
        drain(g, b)
        drain_wb(g - NBUF, b)
        transpose(b)
        fire_wb(g, b)
    for b in range(NBUF):
        drain_wb(N_SLOTS - NBUF + b, b)


def kernel(sentences_indices, table):
    idx_t = sentences_indices.T.astype(jnp.int32).reshape(T, SB, 128)
    out_p = _emb_lookup(idx_t, table)
    out5 = out_p.reshape(T, D // 8, SB, 8, 128)
    return out5.transpose(2, 4, 0, 1, 3).reshape(S, T, D)


# speedup vs baseline: 1.6213x; 1.3855x over previous
import functools

import jax
import jax.numpy as jnp
from jax import lax
from jax.experimental import pallas as pl
from jax.experimental.pallas import tpu as pltpu
from jax.experimental.pallas import tpu_sc as plsc

NUM_CORES = 2
NUM_SUBCORES = 16
NUM_WORKERS = NUM_CORES * NUM_SUBCORES

S = 4096
T = 200
D = 64
SB = S // 128
L = 16

GT = 2
NBUF = 2
N_SLOTS = T // GT

_mesh = plsc.VectorSubcoreMesh(core_axis_name="c", subcore_axis_name="s")


@functools.partial(
    pl.kernel,
    mesh=_mesh,
    out_type=jax.ShapeDtypeStruct((T, D // 8, SB, 1024), jnp.float32),
    scratch_types=[
        pltpu.VMEM((T, 128), jnp.int32),
        pltpu.VMEM((GT * 128, D), jnp.float32),
        pltpu.VMEM((GT * 128, D), jnp.float32),
        pltpu.VMEM((GT * 128 * D,), jnp.float32),
        pltpu.VMEM((GT * 128 * D,), jnp.float32),
        pltpu.SemaphoreType.DMA,
        pltpu.SemaphoreType.DMA,
        pltpu.SemaphoreType.DMA,
        pltpu.SemaphoreType.DMA,
    ],
    compiler_params=pltpu.CompilerParams(
        use_tc_tiling_on_sc=False,
        needs_layout_passes=False,
    ),
)
def _emb_lookup(idx_hbm, table_hbm, out_hbm, idx_v, rows0, rows1,
                tp0, tp1, gat_sem0, gat_sem1, wb_sem0, wb_sem1):
    rows = (rows0, rows1)
    tp = (tp0, tp1)
    gat_sem = (gat_sem0, gat_sem1)
    wb_sem = (wb_sem0, wb_sem1)

    wid = lax.axis_index("s") * NUM_CORES + lax.axis_index("c")

    pltpu.async_copy(idx_hbm.at[:, wid, :], idx_v, gat_sem0).wait()

    def gathers(g, b):
        out = []
        for tt in range(GT):
            out.append((
                table_hbm.at[idx_v.at[g * GT + tt]],
                rows[b].at[pl.ds(tt * 128, 128)],
                gat_sem[b],
            ))
        return out

    def fire(g, b):
        for src, dst, sem in gathers(g, b):
            pltpu.async_copy(src, dst, sem)

    def drain(g, b):
        for src, dst, sem in gathers(g, b):
            pltpu.make_async_copy(src, dst, sem).wait()

    iota = lax.iota(jnp.int32, L)

    pvec = [((k * L + iota) // 8) * 1024 + ((k * L + iota) % 8) * 128
            for k in range(D // L)]

    def transpose(b):
        def row_body(r):
            base = (r // 128) * 8192 + (r % 128)
            for k in range(D // L):
                v = plsc.load_gather(
                    rows[b], [iota * 0 + r, iota + k * L])
                plsc.store_scatter(tp[b], [pvec[k] + base], v)

        plsc.parallel_loop(0, GT * 128, 1, unroll=8)(row_body)

    def wb_copies(g, b):
        out = []
        for tt in range(GT):
            for db in range(D // 8):
                out.append((
                    tp[b].at[pl.ds((tt * 8 + db) * 1024, 1024)],
                    out_hbm.at[g * GT + tt, db, wid],
                    wb_sem[b],
                ))
        return out

    def fire_wb(g, b):
        for src, dst, sem in wb_copies(g, b):
            pltpu.async_copy(src, dst, sem)

    def drain_wb(g, b):
        for src, dst, sem in wb_copies(g, b):
            pltpu.make_async_copy(src, dst, sem).wait()

    for b in range(NBUF):
        fire(b, b)

    def body(k, _):
        for b in range(NBUF):
            g = k * NBUF + b
            drain(g, b)
            @pl.when(k > 0)
            def _():
                drain_wb(g - NBUF, b)
            transpose(b)
            fire_wb(g, b)
            fire(g + NBUF, b)
        return ()

    lax.fori_loop(0, N_SLOTS // NBUF - 1, body, (), unroll=False)

    for b in range(NBUF):
        g = N_SLOTS - NBUF + b
        drain(g, b)
        drain_wb(g - NBUF, b)
        transpose(b)
        fire_wb(g, b)
    for b in range(NBUF):
        drain_wb(N_SLOTS - NBUF + b, b)


def kernel(sentences_indices, table):
    idx_t = sentences_indices.T.astype(jnp.int32).reshape(T, SB, 128)
    out_p = _emb_lookup(idx_t, table)
    out5 = out_p.reshape(T, D // 8, SB, 8, 128)
    return out5.transpose(2, 4, 0, 1, 3).reshape(S, T, D)


# speedup vs baseline: 1.6437x; 1.0139x over previous
import functools

import jax
import jax.numpy as jnp
from jax import lax
from jax.experimental import pallas as pl
from jax.experimental.pallas import tpu as pltpu
from jax.experimental.pallas import tpu_sc as plsc

NUM_CORES = 2
NUM_SUBCORES = 16
NUM_WORKERS = NUM_CORES * NUM_SUBCORES

S = 4096
T = 200
D = 64
SB = S // 128
L = 16

GT = 2
NBUF = 2
N_SLOTS = T // GT

_mesh = plsc.VectorSubcoreMesh(core_axis_name="c", subcore_axis_name="s")


@functools.partial(
    pl.kernel,
    mesh=_mesh,
    out_type=jax.ShapeDtypeStruct((T, D // 8, SB, 1024), jnp.float32),
    scratch_types=[
        pltpu.VMEM((T, 128), jnp.int32),
        pltpu.VMEM((GT * 128, D), jnp.float32),
        pltpu.VMEM((GT * 128, D), jnp.float32),
        pltpu.VMEM((GT * 128 * D,), jnp.float32),
        pltpu.VMEM((GT * 128 * D,), jnp.float32),
        pltpu.SemaphoreType.DMA,
        pltpu.SemaphoreType.DMA,
        pltpu.SemaphoreType.DMA,
        pltpu.SemaphoreType.DMA,
    ],
    compiler_params=pltpu.CompilerParams(
        use_tc_tiling_on_sc=False,
        needs_layout_passes=False,
    ),
)
def _emb_lookup(idx_hbm, table_hbm, out_hbm, idx_v, rows0, rows1,
                tp0, tp1, gat_sem0, gat_sem1, wb_sem0, wb_sem1):
    rows = (rows0, rows1)
    tp = (tp0, tp1)
    gat_sem = (gat_sem0, gat_sem1)
    wb_sem = (wb_sem0, wb_sem1)

    wid = lax.axis_index("s") * NUM_CORES + lax.axis_index("c")

    pltpu.async_copy(idx_hbm.at[:, wid, :], idx_v, gat_sem0).wait()

    def gathers(g, b):
        out = []
        for tt in range(GT):
            out.append((
                table_hbm.at[idx_v.at[g * GT + tt]],
                rows[b].at[pl.ds(tt * 128, 128)],
                gat_sem[b],
            ))
        return out

    def fire(g, b):
        for src, dst, sem in gathers(g, b):
            pltpu.async_copy(src, dst, sem)

    def drain(g, b):
        for src, dst, sem in gathers(g, b):
            pltpu.make_async_copy(src, dst, sem).wait()

    iota = lax.iota(jnp.int32, L)

    pvec = [((k * L + iota) // 8) * 1024 + ((k * L + iota) % 8) * 128
            for k in range(D // L)]

    def transpose(b):
        def row_body(r):
            bvec = iota * 0 + ((r // 128) * 8192 + (r % 128))
            for k in range(D // L):
                v = rows[b][r, pl.ds(k * L, L)]
                plsc.store_scatter(tp[b], [pvec[k] + bvec], v)

        plsc.parallel_loop(0, GT * 128, 1, unroll=8)(row_body)

    def wb_copies(g, b):
        out = []
        for tt in range(GT):
            for db in range(D // 8):
                out.append((
                    tp[b].at[pl.ds((tt * 8 + db) * 1024, 1024)],
                    out_hbm.at[g * GT + tt, db, wid],
                    wb_sem[b],
                ))
        return out

    def fire_wb(g, b):
        for src, dst, sem in wb_copies(g, b):
            pltpu.async_copy(src, dst, sem)

    def drain_wb(g, b):
        for src, dst, sem in wb_copies(g, b):
            pltpu.make_async_copy(src, dst, sem).wait()

    for b in range(NBUF):
        fire(b, b)

    def body(k, _):
        for b in range(NBUF):
            g = k * NBUF + b
            drain(g, b)
            @pl.when(k > 0)
            def _():
                drain_wb(g - NBUF, b)
            transpose(b)
            fire_wb(g, b)
            fire(g + NBUF, b)
        return ()

    lax.fori_loop(0, N_SLOTS // NBUF - 1, body, (), unroll=False)

    for b in range(NBUF):
        g = N_SLOTS - NBUF + b
        drain(g, b)
        drain_wb(g - NBUF, b)
        transpose(b)
        fire_wb(g, b)
    for b in range(NBUF):
        drain_wb(N_SLOTS - NBUF + b, b)


def kernel(sentences_indices, table):
    idx_t = sentences_indices.T.astype(jnp.int32).reshape(T, SB, 128)
    out_p = _emb_lookup(idx_t, table)
    out5 = out_p.reshape(T, D // 8, SB, 8, 128)
    return out5.transpose(2, 4, 0, 1, 3).reshape(S, T, D)


# speedup vs baseline: 2.1375x; 1.3004x over previous
import functools

import jax
import jax.numpy as jnp
from jax import lax
from jax.experimental import pallas as pl
from jax.experimental.pallas import tpu as pltpu
from jax.experimental.pallas import tpu_sc as plsc

NUM_CORES = 2
NUM_SUBCORES = 16
NUM_WORKERS = NUM_CORES * NUM_SUBCORES

S = 4096
T = 200
B = S * T
D = 64

STREAM = 100
GROUP = 400
G = GROUP // STREAM

B_PER_W = B // NUM_WORKERS
N_GROUPS = B_PER_W // GROUP
ROWS_PER_W = B_PER_W // STREAM

NBUF = 2

_mesh = plsc.VectorSubcoreMesh(core_axis_name="c", subcore_axis_name="s")


@functools.partial(
    pl.kernel,
    mesh=_mesh,
    out_type=jax.ShapeDtypeStruct((B, 2 * D), jnp.float32),
    scratch_types=[
        pltpu.VMEM((ROWS_PER_W, STREAM), jnp.int32),
        pltpu.VMEM((GROUP, 2 * D), jnp.float32),
        pltpu.VMEM((GROUP, 2 * D), jnp.float32),
        pltpu.SemaphoreType.DMA,
        pltpu.SemaphoreType.DMA,
        pltpu.SemaphoreType.DMA,
        pltpu.SemaphoreType.DMA,
    ],
    compiler_params=pltpu.CompilerParams(use_tc_tiling_on_sc=False),
)
def _emb_lookup(idx_hbm, table_hbm, out_hbm, idx_v, rows0, rows1,
                gat_sem0, gat_sem1, wb_sem0, wb_sem1):
    rows = (rows0, rows1)
    gat_sem = (gat_sem0, gat_sem1)
    wb_sem = (wb_sem0, wb_sem1)

    wid = lax.axis_index("s") * NUM_CORES + lax.axis_index("c")
    out_base = wid * B_PER_W

    pltpu.async_copy(
        idx_hbm.at[pl.ds(wid * ROWS_PER_W, ROWS_PER_W)], idx_v, gat_sem0
    ).wait()

    def streams(g, b):
        out = []
        for j in range(G):
            out.append((
                table_hbm.at[idx_v.at[g * G + j]],
                rows[b].at[pl.ds(j * STREAM, STREAM)],
                gat_sem[b],
            ))
        return out

    def fire(g, b):
        for src, dst, sem in streams(g, b):
            pltpu.async_copy(src, dst, sem)

    def drain_gathers(g, b):
        for src, dst, sem in streams(g, b):
            pltpu.make_async_copy(src, dst, sem).wait()

    for b in range(NBUF):
        fire(b, b)

    def body(k, _):
        for b in range(NBUF):
            g_done = k * NBUF + b
            drain_gathers(g_done, b)
            pltpu.async_copy(
                rows[b], out_hbm.at[pl.ds(out_base + g_done * GROUP, GROUP)],
                wb_sem[b],
            ).wait()
            fire(g_done + NBUF, b)
        return ()

    lax.fori_loop(0, N_GROUPS // NBUF - 1, body, (), unroll=False)

    for b in range(NBUF):
        g_done = N_GROUPS - NBUF + b
        drain_gathers(g_done, b)
        pltpu.async_copy(
            rows[b], out_hbm.at[pl.ds(out_base + g_done * GROUP, GROUP)],
            wb_sem[b],
        ).wait()


def kernel(sentences_indices, table):
    idx2d = sentences_indices.reshape(B // STREAM, STREAM).astype(jnp.int32)
    table128 = jnp.pad(table, ((0, 0), (0, D)))
    out_p = _emb_lookup(idx2d, table128)
    return out_p[:, :D].reshape(S, T, D)


# speedup vs baseline: 2.1402x; 1.0013x over previous
import functools

import jax
import jax.numpy as jnp
from jax import lax
from jax.experimental import pallas as pl
from jax.experimental.pallas import tpu as pltpu
from jax.experimental.pallas import tpu_sc as plsc

NUM_CORES = 2
NUM_SUBCORES = 16
NUM_WORKERS = NUM_CORES * NUM_SUBCORES

S = 4096
T = 200
B = S * T
D = 64

STREAM = 128
GROUP = 256
G = GROUP // STREAM

B_PER_W = B // NUM_WORKERS
N_GROUPS = B_PER_W // GROUP
ROWS_PER_W = B_PER_W // STREAM

NBUF = 2

_mesh = plsc.VectorSubcoreMesh(core_axis_name="c", subcore_axis_name="s")


@functools.partial(
    pl.kernel,
    mesh=_mesh,
    out_type=jax.ShapeDtypeStruct((B, 2 * D), jnp.float32),
    scratch_types=[
        pltpu.VMEM((ROWS_PER_W, STREAM), jnp.int32),
        pltpu.VMEM((GROUP, 2 * D), jnp.float32),
        pltpu.VMEM((GROUP, 2 * D), jnp.float32),
        pltpu.SemaphoreType.DMA,
        pltpu.SemaphoreType.DMA,
        pltpu.SemaphoreType.DMA,
        pltpu.SemaphoreType.DMA,
    ],
    compiler_params=pltpu.CompilerParams(use_tc_tiling_on_sc=False),
)
def _emb_lookup(idx_hbm, table_hbm, out_hbm, idx_v, rows0, rows1,
                gat_sem0, gat_sem1, wb_sem0, wb_sem1):
    rows = (rows0, rows1)
    gat_sem = (gat_sem0, gat_sem1)
    wb_sem = (wb_sem0, wb_sem1)

    wid = lax.axis_index("s") * NUM_CORES + lax.axis_index("c")
    out_base = wid * B_PER_W

    pltpu.async_copy(
        idx_hbm.at[pl.ds(wid * ROWS_PER_W, ROWS_PER_W)], idx_v, gat_sem0
    ).wait()

    def streams(g, b):
        out = []
        for j in range(G):
            out.append((
                table_hbm.at[idx_v.at[g * G + j]],
                rows[b].at[pl.ds(j * STREAM, STREAM)],
                gat_sem[b],
            ))
        return out

    def fire(g, b):
        for src, dst, sem in streams(g, b):
            pltpu.async_copy(src, dst, sem)

    def drain_gathers(g, b):
        for src, dst, sem in streams(g, b):
            pltpu.make_async_copy(src, dst, sem).wait()

    for b in range(NBUF):
        fire(b, b)

    def body(k, _):
        for b in range(NBUF):
            g_done = k * NBUF + b
            drain_gathers(g_done, b)
            pltpu.async_copy(
                rows[b], out_hbm.at[pl.ds(out_base + g_done * GROUP, GROUP)],
                wb_sem[b],
            ).wait()
            fire(g_done + NBUF, b)
        return ()

    lax.fori_loop(0, N_GROUPS // NBUF - 1, body, (), unroll=False)

    for b in range(NBUF):
        g_done = N_GROUPS - NBUF + b
        drain_gathers(g_done, b)
        pltpu.async_copy(
            rows[b], out_hbm.at[pl.ds(out_base + g_done * GROUP, GROUP)],
            wb_sem[b],
        ).wait()


def kernel(sentences_indices, table):
    idx2d = sentences_indices.reshape(B // STREAM, STREAM).astype(jnp.int32)
    table128 = jnp.pad(table, ((0, 0), (0, D)))
    out_p = _emb_lookup(idx2d, table128)
    return out_p[:, :D].reshape(S, T, D)
